# baseline (speedup 1.0000x reference)
import functools

import jax
import jax.numpy as jnp
from jax import lax
from jax.experimental import pallas as pl
from jax.experimental.pallas import tpu as pltpu
from jax.experimental.pallas import tpu_sc as plsc

BATCH = 4096
HIST = 200
EMBED_DIM = 32

_info = plsc.get_sparse_core_info()
_NC, _NS = _info.num_cores, _info.num_subcores
_NW = _NC * _NS

_B = BATCH * HIST
_BW = BATCH // _NW
_B_PER_W = _BW * HIST
_CHUNK = 800
_STEPS = _B_PER_W // _CHUNK
_HFULL = HIST // 16
_HTAIL = HIST - _HFULL * 16


def _make_gather(V, D):
    mesh = plsc.VectorSubcoreMesh(core_axis_name="c", subcore_axis_name="s")

    @functools.partial(
        pl.kernel,
        mesh=mesh,
        compiler_params=pltpu.CompilerParams(
            use_tc_tiling_on_sc=False, needs_layout_passes=False),
        out_type=jax.ShapeDtypeStruct((_B, D), jnp.float32),
        scratch_types=[
            pltpu.VMEM((HIST, _BW), jnp.int32),
            pltpu.VMEM((_B_PER_W,), jnp.int32),
            pltpu.VMEM((_CHUNK, D), jnp.float32),
            pltpu.VMEM((_CHUNK, D), jnp.float32),
            pltpu.SemaphoreType.DMA,
            pltpu.SemaphoreType.DMA,
            pltpu.SemaphoreType.DMA,
            pltpu.SemaphoreType.DMA,
        ],
    )
    def gather_kernel(table_hbm, idxt_hbm, out_hbm, slab_v, idx_v, rows0,
                      rows1, gsem0, gsem1, ssem0, ssem1):
        wid = lax.axis_index("s") * _NC + lax.axis_index("c")
        wbase = wid * _B_PER_W
        rows = (rows0, rows1)
        gsem = (gsem0, gsem1)
        ssem = (ssem0, ssem1)

        pltpu.sync_copy(idxt_hbm.at[:, pl.ds(wid * _BW, _BW)], slab_v)

        iota = lax.iota(jnp.int32, 16)

        def transpose_body(k, carry):
            p = iota + k * 16
            h = p % HIST
            b = p // HIST
            idx_v[pl.ds(k * 16, 16)] = plsc.load_gather(slab_v, [h, b])
            return carry

        lax.fori_loop(0, _B_PER_W // 16, transpose_body, 0)

        def gather_of(i, b):
            return pltpu.make_async_copy(
                table_hbm.at[idx_v.at[pl.ds(i * _CHUNK, _CHUNK)]],
                rows[b], gsem[b])

        def store_of(i, b):
            return pltpu.make_async_copy(
                rows[b], out_hbm.at[pl.ds(wbase + i * _CHUNK, _CHUNK)],
                ssem[b])

        gather_of(0, 0).start()

        def outer(g, carry):
            for b in range(2):
                i = 2 * g + b
                nb = 1 - b
                gather_of(i, b).wait()

                @pl.when(i >= 1)
                def _():
                    store_of(i - 1, nb).wait()

                @pl.when(i + 1 < _STEPS)
                def _():
                    gather_of(i + 1, nb).start()

                store_of(i, b).start()
            return carry

        lax.fori_loop(0, _STEPS // 2, outer, 0)
        store_of(_STEPS - 1, (_STEPS - 1) % 2).wait()

    return gather_kernel


def kernel(input, table):
    idx_t = input.T.astype(jnp.int32)
    V, D = table.shape
    out = _make_gather(V, D)(table, idx_t)
    return out.reshape(input.shape[0], input.shape[1], D)
